# baseline (speedup 1.0000x reference)
import jax
import jax.numpy as jnp
import numpy as np
from jax.experimental import pallas as pl
from jax.experimental.pallas import tpu as pltpu

IMB = 8
MROWS = IMB * 1024
PAD = 32
SP = 8


def _cnn_kernel(xblk, xm, ym, w1c, b1, w2c, b2, fcw, fcb, mblk, out,
                x1p, x2p, trip1, trip2):
    c1, c2 = 3, 64
    f32 = jnp.float32

    x1p[SP:SP + MROWS, :] = xblk[...]
    x1p[0:SP, :] = jnp.zeros((SP, c1), f32)
    x1p[SP + MROWS:, :] = jnp.zeros((SP, c1), f32)
    trip1[0:PAD, :] = jnp.zeros((PAD, 3 * c1), f32)
    trip1[PAD + MROWS:, :] = jnp.zeros((PAD, 3 * c1), f32)
    for k in range(3):
        ox = k - 1
        trip1[PAD:PAD + MROWS, k * c1:(k + 1) * c1] = (
            x1p[SP + ox:SP + ox + MROWS, :] * xm[:, k:k + 1])

    acc1 = jnp.zeros((MROWS, 64), f32)
    for j in range(3):
        oy = j - 1
        t = jnp.dot(trip1[PAD + 32 * oy:PAD + 32 * oy + MROWS, :], w1c[j],
                    preferred_element_type=f32)
        acc1 = acc1 + t * ym[:, j:j + 1]
    acc1 = jax.nn.relu(acc1 + b1[...])

    x2p[SP:SP + MROWS, :] = acc1
    x2p[0:SP, :] = jnp.zeros((SP, c2), f32)
    x2p[SP + MROWS:, :] = jnp.zeros((SP, c2), f32)
    trip2[0:PAD, :] = jnp.zeros((PAD, 3 * c2), f32)
    trip2[PAD + MROWS:, :] = jnp.zeros((PAD, 3 * c2), f32)
    for k in range(3):
        ox = k - 1
        trip2[PAD:PAD + MROWS, k * c2:(k + 1) * c2] = (
            x2p[SP + ox:SP + ox + MROWS, :] * xm[:, k:k + 1])

    acc2 = jnp.zeros((MROWS, 128), f32)
    for j in range(3):
        oy = j - 1
        t = jnp.dot(trip2[PAD + 32 * oy:PAD + 32 * oy + MROWS, :], w2c[j],
                    preferred_element_type=f32)
        acc2 = acc2 + t * ym[:, j:j + 1]
    acc2 = jax.nn.relu(acc2 + b2[...])

    pooled = jnp.mean(acc2.reshape(IMB, 1024, 128), axis=1)
    feat = jnp.dot(pooled, fcw[...], preferred_element_type=f32) + fcb[...]
    out[...] = feat * mblk[...]


def _sage_kernel(xg, w1l, b1l, w1r, b1r, w2l, b2l, w2r, b2r, out):
    f32 = jnp.float32
    x = xg[...]
    xr = x.reshape(16, 32, 128)
    s = jnp.sum(xr, axis=1, keepdims=True)
    mean = ((s - xr) * (1.0 / 31.0)).reshape(512, 128)
    h = jax.nn.relu(jnp.dot(mean, w1l[...], preferred_element_type=f32) + b1l[...]
                    + jnp.dot(x, w1r[...], preferred_element_type=f32) + b1r[...])
    hr = h.reshape(16, 32, 128)
    s2 = jnp.sum(hr, axis=1, keepdims=True)
    mean2 = ((s2 - hr) * (1.0 / 31.0)).reshape(512, 128)
    out[...] = (jnp.dot(mean2, w2l[...], preferred_element_type=f32) + b2l[...]
                + jnp.dot(h, w2r[...], preferred_element_type=f32) + b2r[...])


def _border_masks():
    m = np.arange(1024)
    xcol = m % 32
    yrow = m // 32
    xm = np.stack([((xcol + ox) >= 0) & ((xcol + ox) < 32) for ox in (-1, 0, 1)],
                  axis=1).astype(np.float32)
    ym = np.stack([((yrow + oy) >= 0) & ((yrow + oy) < 32) for oy in (-1, 0, 1)],
                  axis=1).astype(np.float32)
    return (jnp.asarray(np.tile(xm, (IMB, 1))), jnp.asarray(np.tile(ym, (IMB, 1))))


def kernel(x, mask, conv1_w, conv1_b, conv2_w, conv2_b, fc_w, fc_b,
           s1_wl, s1_bl, s1_wr, s1_br, s2_wl, s2_bl, s2_wr, s2_br):
    batch, cars, c, h, w = x.shape
    n_img = batch * cars
    xr = jnp.transpose(x, (0, 1, 3, 4, 2)).reshape(n_img * h * w, c)
    w1c = jnp.transpose(conv1_w, (2, 3, 1, 0)).reshape(3, 9, 64)
    w2c = jnp.transpose(conv2_w, (2, 3, 1, 0)).reshape(3, 192, 128)
    xmask, ymask = _border_masks()
    grid = n_img // IMB

    full = lambda a: pl.BlockSpec(a.shape, lambda i: (0,) * a.ndim)
    b1 = conv1_b.reshape(1, 64)
    b2 = conv2_b.reshape(1, 128)
    fcb = fc_b.reshape(1, 128)
    mflat = mask.reshape(n_img, 1)

    feats = pl.pallas_call(
        _cnn_kernel,
        grid=(grid,),
        in_specs=[
            pl.BlockSpec((MROWS, c), lambda i: (i, 0)),
            full(xmask), full(ymask),
            full(w1c), full(b1), full(w2c), full(b2),
            full(fc_w), full(fcb),
            pl.BlockSpec((IMB, 1), lambda i: (i, 0)),
        ],
        out_specs=pl.BlockSpec((IMB, 128), lambda i: (i, 0)),
        out_shape=jax.ShapeDtypeStruct((n_img, 128), jnp.float32),
        scratch_shapes=[
            pltpu.VMEM((MROWS + 2 * SP, 3), jnp.float32),
            pltpu.VMEM((MROWS + 2 * SP, 64), jnp.float32),
            pltpu.VMEM((MROWS + 2 * PAD, 9), jnp.float32),
            pltpu.VMEM((MROWS + 2 * PAD, 192), jnp.float32),
        ],
        compiler_params=pltpu.CompilerParams(
            dimension_semantics=("parallel",)),
    )(xr, xmask, ymask, w1c, b1, w2c, b2, fc_w.T, fcb, mflat)

    sage_in = (feats, s1_wl.T, s1_bl.reshape(1, 128), s1_wr.T,
               s1_br.reshape(1, 128), s2_wl.T, s2_bl.reshape(1, 128),
               s2_wr.T, s2_br.reshape(1, 128))
    res = pl.pallas_call(
        _sage_kernel,
        grid=(1,),
        in_specs=[full(a) for a in sage_in],
        out_specs=pl.BlockSpec((n_img, 128), lambda i: (0, 0)),
        out_shape=jax.ShapeDtypeStruct((n_img, 128), jnp.float32),
    )(*sage_in)

    return res.reshape(batch, cars, 128)
